# baseline (speedup 1.0000x reference)
import functools

import jax
import jax.numpy as jnp
from jax import lax
from jax.experimental import pallas as pl
from jax.experimental.pallas import tpu as pltpu
from jax.experimental.pallas import tpu_sc as plsc

_CHUNK = 512


@functools.cache
def _make_lookup(B, V, D):
    info = plsc.get_sparse_core_info()
    nc, ns = info.num_cores, info.num_subcores
    nw = nc * ns
    b_per_w = B // nw
    n_chunks = b_per_w // _CHUNK
    mesh = plsc.VectorSubcoreMesh(core_axis_name="c", subcore_axis_name="s")

    @functools.partial(
        pl.kernel,
        out_type=jax.ShapeDtypeStruct((B, D), jnp.float32),
        mesh=mesh,
        scratch_types=[
            pltpu.VMEM((_CHUNK,), jnp.int32),
            pltpu.VMEM((_CHUNK, D), jnp.float32),
            pltpu.SemaphoreType.DMA,
        ],
        compiler_params=pltpu.CompilerParams(use_tc_tiling_on_sc=False),
    )
    def lookup(ids_hbm, table_hbm, out_hbm, idx_v, rows_v, sem):
        wid = lax.axis_index("s") * nc + lax.axis_index("c")
        base = wid * b_per_w

        def body(i, carry):
            off = base + i * _CHUNK
            pltpu.sync_copy(ids_hbm.at[pl.ds(off, _CHUNK)], idx_v)
            pltpu.async_copy(table_hbm.at[idx_v], rows_v, sem).wait()
            pltpu.sync_copy(rows_v, out_hbm.at[pl.ds(off, _CHUNK)])
            return carry

        lax.fori_loop(0, n_chunks, body, 0)

    return lookup


def kernel(token_ids, weights):
    batch, seq = token_ids.shape
    vocab, d = weights.shape
    ids = token_ids.reshape(-1).astype(jnp.int32)
    out = _make_lookup(ids.shape[0], vocab, d)(ids, weights)
    return out.reshape(batch, seq, d)


# speedup vs baseline: 1.0754x; 1.0754x over previous
import functools

import jax
import jax.numpy as jnp
from jax import lax
from jax.experimental import pallas as pl
from jax.experimental.pallas import tpu as pltpu
from jax.experimental.pallas import tpu_sc as plsc

_CHUNK = 800
_NBUF = 2


@functools.cache
def _make_lookup(B, V, D):
    info = plsc.get_sparse_core_info()
    nc, ns = info.num_cores, info.num_subcores
    nw = nc * ns
    b_per_w = B // nw
    n_super = b_per_w // (_CHUNK * _NBUF)
    assert b_per_w == n_super * _CHUNK * _NBUF
    mesh = plsc.VectorSubcoreMesh(core_axis_name="c", subcore_axis_name="s")

    @functools.partial(
        pl.kernel,
        out_type=jax.ShapeDtypeStruct((B, D), jnp.float32),
        mesh=mesh,
        scratch_types=[
            pltpu.VMEM((_NBUF, _CHUNK), jnp.int32),
            pltpu.VMEM((_NBUF, _CHUNK, D), jnp.float32),
            pltpu.SemaphoreType.DMA((_NBUF,)),
            pltpu.SemaphoreType.DMA((_NBUF,)),
        ],
        compiler_params=pltpu.CompilerParams(use_tc_tiling_on_sc=False),
    )
    def lookup(ids_hbm, table_hbm, out_hbm, idx_v, rows_v, gsem, wsem):
        wid = lax.axis_index("s") * nc + lax.axis_index("c")
        base = wid * b_per_w

        def super_body(i, carry):
            for b in range(_NBUF):
                off = base + (i * _NBUF + b) * _CHUNK

                @pl.when(i > 0)
                def _drain(b=b, off=off):
                    pltpu.make_async_copy(
                        rows_v.at[b], out_hbm.at[pl.ds(off, _CHUNK)], wsem.at[b]
                    ).wait()

                pltpu.sync_copy(ids_hbm.at[pl.ds(off, _CHUNK)], idx_v.at[b])
                pltpu.async_copy(table_hbm.at[idx_v.at[b]], rows_v.at[b], gsem.at[b])
            for b in range(_NBUF):
                off = base + (i * _NBUF + b) * _CHUNK
                pltpu.make_async_copy(
                    table_hbm.at[idx_v.at[b]], rows_v.at[b], gsem.at[b]
                ).wait()
                pltpu.async_copy(rows_v.at[b], out_hbm.at[pl.ds(off, _CHUNK)], wsem.at[b])
            return carry

        lax.fori_loop(0, n_super, super_body, 0)
        for b in range(_NBUF):
            pltpu.make_async_copy(
                rows_v.at[b], out_hbm.at[pl.ds(base, _CHUNK)], wsem.at[b]
            ).wait()

    return lookup


def kernel(token_ids, weights):
    batch, seq = token_ids.shape
    vocab, d = weights.shape
    ids = token_ids.reshape(-1).astype(jnp.int32)
    out = _make_lookup(ids.shape[0], vocab, d)(ids, weights)
    return out.reshape(batch, seq, d)
